# baseline (speedup 1.0000x reference)
import functools

import jax
import jax.numpy as jnp
from jax import lax
from jax.experimental import pallas as pl
from jax.experimental.pallas import tpu as pltpu
from jax.experimental.pallas import tpu_sc as plsc

HIDDEN = 256
OUT_DIM = 384
BATCH = 16384
NUM_LANES = 16
LANE_GROUPS = HIDDEN // NUM_LANES

NC = 2
NS = 16
NW = NC * NS
B_PER_W = BATCH // NW
CHUNK = 128
NCHUNK = B_PER_W // CHUNK
NBUF = 3


def _sc_partial_sums(head_idx, rel_idx, node_emb, rel_emb):
  mesh = plsc.VectorSubcoreMesh(core_axis_name="c", subcore_axis_name="s")

  @functools.partial(
      pl.kernel,
      out_type=jax.ShapeDtypeStruct((NW, HIDDEN), jnp.float32),
      mesh=mesh,
      scratch_types=[
          pltpu.VMEM((NCHUNK, CHUNK), jnp.int32),
          pltpu.VMEM((NCHUNK, CHUNK), jnp.int32),
          pltpu.VMEM((CHUNK, HIDDEN), jnp.float32),
          pltpu.VMEM((CHUNK, HIDDEN), jnp.float32),
          pltpu.VMEM((CHUNK, HIDDEN), jnp.float32),
          pltpu.VMEM((HIDDEN,), jnp.float32),
          pltpu.SemaphoreType.DMA,
          pltpu.SemaphoreType.DMA,
          pltpu.SemaphoreType.DMA,
      ],
  )
  def sc_kernel(head_hbm, rel_hbm, node_hbm, relemb_hbm, out_hbm,
                hidx_v, ridx_v, buf_0, buf_1, buf_2, acc_v,
                sem_0, sem_1, sem_2):
    wid = lax.axis_index("s") * NC + lax.axis_index("c")
    pltpu.sync_copy(head_hbm.at[wid], hidx_v)
    pltpu.sync_copy(rel_hbm.at[wid], ridx_v)

    bufs = (buf_0, buf_1, buf_2)
    sems = (sem_0, sem_1, sem_2)
    plan = [(node_hbm, hidx_v, c) for c in range(NCHUNK)] + \
           [(relemb_hbm, ridx_v, c) for c in range(NCHUNK)]

    acc = tuple(jnp.zeros((NUM_LANES,), jnp.float32)
                for _ in range(LANE_GROUPS))

    def accumulate(buf, acc):
      def body(r, acc):
        return tuple(acc[j] + buf[r, pl.ds(j * NUM_LANES, NUM_LANES)]
                     for j in range(LANE_GROUPS))
      return lax.fori_loop(0, CHUNK, body, acc)

    handles = []
    for i in range(NBUF - 1):
      table, idx, c = plan[i]
      handles.append(
          pltpu.async_copy(table.at[idx.at[c]], bufs[i % NBUF],
                           sems[i % NBUF]))
    for i in range(len(plan)):
      if i + NBUF - 1 < len(plan):
        table, idx, c = plan[i + NBUF - 1]
        handles.append(
            pltpu.async_copy(table.at[idx.at[c]],
                             bufs[(i + NBUF - 1) % NBUF],
                             sems[(i + NBUF - 1) % NBUF]))
      handles[i].wait()
      acc = accumulate(bufs[i % NBUF], acc)

    for j in range(LANE_GROUPS):
      acc_v[pl.ds(j * NUM_LANES, NUM_LANES)] = acc[j]
    pltpu.sync_copy(acc_v, out_hbm.at[wid])

  return sc_kernel(head_idx, rel_idx, node_emb, rel_emb)


def _tc_finish(partials, W, b2):
  def body(part_ref, w_ref, b_ref, out_ref):
    pooled = jnp.sum(part_ref[...], axis=0, keepdims=True) * (1.0 / BATCH)
    out_ref[...] = lax.dot_general(
        pooled, w_ref[...], (((1,), (1,)), ((), ())),
        preferred_element_type=jnp.float32) + b_ref[...]

  return pl.pallas_call(
      body,
      out_shape=jax.ShapeDtypeStruct((1, OUT_DIM), jnp.float32),
  )(partials, W, b2)


def kernel(head_index, rel_type, tail_index, node_emb, rel_emb, W, b):
  del tail_index
  h = head_index.astype(jnp.int32).reshape(NW, NCHUNK, CHUNK)
  r = rel_type.astype(jnp.int32).reshape(NW, NCHUNK, CHUNK)
  partials = _sc_partial_sums(h, r, node_emb, rel_emb)
  out = _tc_finish(partials, W, b.reshape(1, OUT_DIM))
  return out.reshape(OUT_DIM)


# speedup vs baseline: 1.2793x; 1.2793x over previous
import functools

import jax
import jax.numpy as jnp
from jax import lax
from jax.experimental import pallas as pl
from jax.experimental.pallas import tpu as pltpu
from jax.experimental.pallas import tpu_sc as plsc

HIDDEN = 256
OUT_DIM = 384
BATCH = 16384
NUM_LANES = 16
LANE_GROUPS = HIDDEN // NUM_LANES
NUM_REL = 1000
RBINS = 1024

NC = 2
NS = 16
NW = NC * NS
B_PER_W = BATCH // NW
CHUNK = 128
NCHUNK = B_PER_W // CHUNK
NBUF = 3
RCOLS = BATCH // RBINS


def _sc_partial_sums(head_idx, node_emb):
  mesh = plsc.VectorSubcoreMesh(core_axis_name="c", subcore_axis_name="s")

  @functools.partial(
      pl.kernel,
      out_type=jax.ShapeDtypeStruct((NW, HIDDEN), jnp.float32),
      mesh=mesh,
      scratch_types=[
          pltpu.VMEM((NCHUNK, CHUNK), jnp.int32),
          pltpu.VMEM((CHUNK, HIDDEN), jnp.float32),
          pltpu.VMEM((CHUNK, HIDDEN), jnp.float32),
          pltpu.VMEM((CHUNK, HIDDEN), jnp.float32),
          pltpu.VMEM((HIDDEN,), jnp.float32),
          pltpu.SemaphoreType.DMA,
          pltpu.SemaphoreType.DMA,
          pltpu.SemaphoreType.DMA,
      ],
  )
  def sc_kernel(head_hbm, node_hbm, out_hbm,
                hidx_v, buf_0, buf_1, buf_2, acc_v, sem_0, sem_1, sem_2):
    wid = lax.axis_index("s") * NC + lax.axis_index("c")
    pltpu.sync_copy(head_hbm.at[wid], hidx_v)

    bufs = (buf_0, buf_1, buf_2)
    sems = (sem_0, sem_1, sem_2)

    acc = tuple(jnp.zeros((NUM_LANES,), jnp.float32)
                for _ in range(LANE_GROUPS))

    def accumulate(buf, acc):
      def body(r, acc):
        return tuple(acc[j] + buf[r, pl.ds(j * NUM_LANES, NUM_LANES)]
                     for j in range(LANE_GROUPS))
      return lax.fori_loop(0, CHUNK, body, acc)

    handles = []
    for i in range(NBUF - 1):
      handles.append(
          pltpu.async_copy(node_hbm.at[hidx_v.at[i]], bufs[i % NBUF],
                           sems[i % NBUF]))
    for i in range(NCHUNK):
      if i + NBUF - 1 < NCHUNK:
        handles.append(
            pltpu.async_copy(node_hbm.at[hidx_v.at[i + NBUF - 1]],
                             bufs[(i + NBUF - 1) % NBUF],
                             sems[(i + NBUF - 1) % NBUF]))
      handles[i].wait()
      acc = accumulate(bufs[i % NBUF], acc)

    for j in range(LANE_GROUPS):
      acc_v[pl.ds(j * NUM_LANES, NUM_LANES)] = acc[j]
    pltpu.sync_copy(acc_v, out_hbm.at[wid])

  return sc_kernel(head_idx, node_emb)


def _tc_rel_pool(rel_ids_t, rel_emb):
  def body(ids_ref, rel_ref, out_ref):
    bins = lax.broadcasted_iota(jnp.int32, (RBINS, RBINS), 1)
    cnt = jnp.zeros((1, RBINS), jnp.float32)
    for t in range(RCOLS):
      col = ids_ref[:, t:t + 1]
      eq = (col == bins).astype(jnp.float32)
      cnt = cnt + jnp.sum(eq, axis=0, keepdims=True)
    out_ref[...] = lax.dot_general(
        cnt[:, :NUM_REL], rel_ref[...], (((1,), (0,)), ((), ())),
        preferred_element_type=jnp.float32)

  return pl.pallas_call(
      body,
      out_shape=jax.ShapeDtypeStruct((1, HIDDEN), jnp.float32),
  )(rel_ids_t, rel_emb)


def _tc_finish(partials, rel_pool, W, b2):
  def body(part_ref, rel_ref, w_ref, b_ref, out_ref):
    pooled = (jnp.sum(part_ref[...], axis=0, keepdims=True)
              + rel_ref[...]) * (1.0 / BATCH)
    out_ref[...] = lax.dot_general(
        pooled, w_ref[...], (((1,), (1,)), ((), ())),
        preferred_element_type=jnp.float32) + b_ref[...]

  return pl.pallas_call(
      body,
      out_shape=jax.ShapeDtypeStruct((1, OUT_DIM), jnp.float32),
  )(partials, rel_pool, W, b2)


def kernel(head_index, rel_type, tail_index, node_emb, rel_emb, W, b):
  del tail_index
  h = head_index.astype(jnp.int32).reshape(NW, NCHUNK, CHUNK)
  r_t = rel_type.astype(jnp.int32).reshape(RCOLS, RBINS).T
  rel_pool = _tc_rel_pool(r_t, rel_emb)
  partials = _sc_partial_sums(h, node_emb)
  out = _tc_finish(partials, rel_pool, W, b.reshape(1, OUT_DIM))
  return out.reshape(OUT_DIM)


# speedup vs baseline: 1.3299x; 1.0396x over previous
import functools

import jax
import jax.numpy as jnp
from jax import lax
from jax.experimental import pallas as pl
from jax.experimental.pallas import tpu as pltpu
from jax.experimental.pallas import tpu_sc as plsc

HIDDEN = 256
OUT_DIM = 384
BATCH = 16384
NUM_LANES = 16
LANE_GROUPS = HIDDEN // NUM_LANES
NUM_REL = 1000
RBINS = 1024

NC = 2
NS = 16
NW = NC * NS
B_PER_W = BATCH // NW
CHUNK = 128
NCHUNK = B_PER_W // CHUNK
NBUF = 3
RCOLS = BATCH // RBINS


def _sc_partial_sums(head_idx, node_emb):
  mesh = plsc.VectorSubcoreMesh(core_axis_name="c", subcore_axis_name="s")

  @functools.partial(
      pl.kernel,
      out_type=jax.ShapeDtypeStruct((NW, HIDDEN), jnp.float32),
      mesh=mesh,
      scratch_types=[
          pltpu.VMEM((NCHUNK, CHUNK), jnp.int32),
          pltpu.VMEM((CHUNK, HIDDEN), jnp.float32),
          pltpu.VMEM((CHUNK, HIDDEN), jnp.float32),
          pltpu.VMEM((CHUNK, HIDDEN), jnp.float32),
          pltpu.VMEM((HIDDEN,), jnp.float32),
          pltpu.SemaphoreType.DMA,
          pltpu.SemaphoreType.DMA,
          pltpu.SemaphoreType.DMA,
      ],
  )
  def sc_kernel(head_hbm, node_hbm, out_hbm,
                hidx_v, buf_0, buf_1, buf_2, acc_v, sem_0, sem_1, sem_2):
    wid = lax.axis_index("s") * NC + lax.axis_index("c")
    pltpu.sync_copy(head_hbm.at[wid], hidx_v)

    bufs = (buf_0, buf_1, buf_2)
    sems = (sem_0, sem_1, sem_2)

    acc = tuple(jnp.zeros((NUM_LANES,), jnp.float32)
                for _ in range(LANE_GROUPS))

    def accumulate(buf, acc):
      def body(r, acc):
        return tuple(acc[j] + buf[r, pl.ds(j * NUM_LANES, NUM_LANES)]
                     for j in range(LANE_GROUPS))
      return lax.fori_loop(0, CHUNK, body, acc)

    handles = []
    for i in range(NBUF - 1):
      handles.append(
          pltpu.async_copy(node_hbm.at[hidx_v.at[i]], bufs[i % NBUF],
                           sems[i % NBUF]))
    for i in range(NCHUNK):
      if i + NBUF - 1 < NCHUNK:
        handles.append(
            pltpu.async_copy(node_hbm.at[hidx_v.at[i + NBUF - 1]],
                             bufs[(i + NBUF - 1) % NBUF],
                             sems[(i + NBUF - 1) % NBUF]))
      handles[i].wait()
      acc = accumulate(bufs[i % NBUF], acc)

    for j in range(LANE_GROUPS):
      acc_v[pl.ds(j * NUM_LANES, NUM_LANES)] = acc[j]
    pltpu.sync_copy(acc_v, out_hbm.at[wid])

  return sc_kernel(head_idx, node_emb)


def _tc_rel_pool(rel_ids, rel_emb):
  def body(ids_ref, rel_ref, out_ref):
    bins = lax.broadcasted_iota(jnp.int32, (RBINS, RBINS), 0)
    cnt = jnp.zeros((RBINS, 1), jnp.float32)
    for t in range(RCOLS):
      row = ids_ref[t:t + 1, :]
      eq = (row == bins).astype(jnp.float32)
      cnt = cnt + jnp.sum(eq, axis=1, keepdims=True)
    out_ref[...] = lax.dot_general(
        cnt[:NUM_REL, :], rel_ref[...], (((0,), (0,)), ((), ())),
        preferred_element_type=jnp.float32)

  return pl.pallas_call(
      body,
      out_shape=jax.ShapeDtypeStruct((1, HIDDEN), jnp.float32),
  )(rel_ids, rel_emb)


def _tc_finish(partials, rel_pool, W, b):
  def body(part_ref, rel_ref, w_ref, b_ref, out_ref):
    pooled = (jnp.sum(part_ref[...], axis=0, keepdims=True)
              + rel_ref[...]) * (1.0 / BATCH)
    res = lax.dot_general(
        pooled, w_ref[...], (((1,), (1,)), ((), ())),
        preferred_element_type=jnp.float32)
    out_ref[...] = res[0] + b_ref[...]

  return pl.pallas_call(
      body,
      out_shape=jax.ShapeDtypeStruct((OUT_DIM,), jnp.float32),
  )(partials, rel_pool, W, b)


def kernel(head_index, rel_type, tail_index, node_emb, rel_emb, W, b):
  del tail_index
  h = head_index.astype(jnp.int32).reshape(NW, NCHUNK, CHUNK)
  r2 = rel_type.astype(jnp.int32).reshape(RCOLS, RBINS)
  rel_pool = _tc_rel_pool(r2, rel_emb)
  partials = _sc_partial_sums(h, node_emb)
  return _tc_finish(partials, rel_pool, W, b)


# speedup vs baseline: 1.3772x; 1.0356x over previous
import functools

import jax
import jax.numpy as jnp
from jax import lax
from jax.experimental import pallas as pl
from jax.experimental.pallas import tpu as pltpu
from jax.experimental.pallas import tpu_sc as plsc

HIDDEN = 256
OUT_DIM = 384
BATCH = 16384
NUM_LANES = 16
LANE_GROUPS = HIDDEN // NUM_LANES
NUM_REL = 1000
RBINS = 1024

NC = 2
NS = 16
NW = NC * NS
B_PER_W = BATCH // NW
CHUNK = 64
NCHUNK = B_PER_W // CHUNK
NBUF = 4
RCOLS = BATCH // RBINS


def _sc_partial_sums(head_idx, node_emb):
  mesh = plsc.VectorSubcoreMesh(core_axis_name="c", subcore_axis_name="s")

  @functools.partial(
      pl.kernel,
      out_type=jax.ShapeDtypeStruct((NW, HIDDEN), jnp.float32),
      mesh=mesh,
      scratch_types=[
          pltpu.VMEM((NCHUNK, CHUNK), jnp.int32),
          pltpu.VMEM((CHUNK, HIDDEN), jnp.float32),
          pltpu.VMEM((CHUNK, HIDDEN), jnp.float32),
          pltpu.VMEM((CHUNK, HIDDEN), jnp.float32),
          pltpu.VMEM((CHUNK, HIDDEN), jnp.float32),
          pltpu.VMEM((HIDDEN,), jnp.float32),
          pltpu.SemaphoreType.DMA,
          pltpu.SemaphoreType.DMA,
          pltpu.SemaphoreType.DMA,
          pltpu.SemaphoreType.DMA,
      ],
  )
  def sc_kernel(head_hbm, node_hbm, out_hbm,
                hidx_v, buf_0, buf_1, buf_2, buf_3, acc_v,
                sem_0, sem_1, sem_2, sem_3):
    wid = lax.axis_index("s") * NC + lax.axis_index("c")
    pltpu.sync_copy(head_hbm.at[wid], hidx_v)

    bufs = (buf_0, buf_1, buf_2, buf_3)
    sems = (sem_0, sem_1, sem_2, sem_3)

    acc = tuple(jnp.zeros((NUM_LANES,), jnp.float32)
                for _ in range(LANE_GROUPS))

    def accumulate(buf, acc):
      def body(r, acc):
        return tuple(acc[j] + buf[r, pl.ds(j * NUM_LANES, NUM_LANES)]
                     for j in range(LANE_GROUPS))
      return lax.fori_loop(0, CHUNK, body, acc)

    handles = []
    for i in range(NBUF - 1):
      handles.append(
          pltpu.async_copy(node_hbm.at[hidx_v.at[i]], bufs[i % NBUF],
                           sems[i % NBUF]))
    for i in range(NCHUNK):
      if i + NBUF - 1 < NCHUNK:
        handles.append(
            pltpu.async_copy(node_hbm.at[hidx_v.at[i + NBUF - 1]],
                             bufs[(i + NBUF - 1) % NBUF],
                             sems[(i + NBUF - 1) % NBUF]))
      handles[i].wait()
      acc = accumulate(bufs[i % NBUF], acc)

    for j in range(LANE_GROUPS):
      acc_v[pl.ds(j * NUM_LANES, NUM_LANES)] = acc[j]
    pltpu.sync_copy(acc_v, out_hbm.at[wid])

  return sc_kernel(head_idx, node_emb)


def _tc_rel_pool(rel_ids, rel_emb):
  def body(ids_ref, rel_ref, out_ref):
    bins = lax.broadcasted_iota(jnp.int32, (RBINS, RBINS), 0)
    cnt = jnp.zeros((RBINS, 1), jnp.float32)
    for t in range(RCOLS):
      row = ids_ref[t:t + 1, :]
      eq = (row == bins).astype(jnp.float32)
      cnt = cnt + jnp.sum(eq, axis=1, keepdims=True)
    out_ref[...] = lax.dot_general(
        cnt[:NUM_REL, :], rel_ref[...], (((0,), (0,)), ((), ())),
        preferred_element_type=jnp.float32)

  return pl.pallas_call(
      body,
      out_shape=jax.ShapeDtypeStruct((1, HIDDEN), jnp.float32),
  )(rel_ids, rel_emb)


def _tc_finish(partials, rel_pool, W, b):
  def body(part_ref, rel_ref, w_ref, b_ref, out_ref):
    pooled = (jnp.sum(part_ref[...], axis=0, keepdims=True)
              + rel_ref[...]) * (1.0 / BATCH)
    res = lax.dot_general(
        pooled, w_ref[...], (((1,), (1,)), ((), ())),
        preferred_element_type=jnp.float32)
    out_ref[...] = res[0] + b_ref[...]

  return pl.pallas_call(
      body,
      out_shape=jax.ShapeDtypeStruct((OUT_DIM,), jnp.float32),
  )(partials, rel_pool, W, b)


def kernel(head_index, rel_type, tail_index, node_emb, rel_emb, W, b):
  del tail_index
  h = head_index.astype(jnp.int32).reshape(NW, NCHUNK, CHUNK)
  r2 = rel_type.astype(jnp.int32).reshape(RCOLS, RBINS)
  rel_pool = _tc_rel_pool(r2, rel_emb)
  partials = _sc_partial_sums(h, node_emb)
  return _tc_finish(partials, rel_pool, W, b)
